# baseline (speedup 1.0000x reference)
import functools

import jax
import jax.numpy as jnp
from jax import lax
from jax.experimental import pallas as pl
from jax.experimental.pallas import tpu as pltpu
from jax.experimental.pallas import tpu_sc as plsc

NC = 2
NS = 16
NW = NC * NS

T = 200
B = 16384
H = 32
TR = T // 8
JB = B // 128
JPW = JB // NW
HB = H // 8


@jax.jit
def _embed_lookup(idx4, table):
    mesh = plsc.VectorSubcoreMesh(core_axis_name="c", subcore_axis_name="s")

    @functools.partial(
        pl.kernel,
        out_type=jax.ShapeDtypeStruct((T, HB, JB * 1024), jnp.float32),
        mesh=mesh,
        scratch_types=[
            pltpu.VMEM((8, 128), jnp.int32),
            pltpu.VMEM((2, 4, 128, H), jnp.float32),
            pltpu.VMEM((2, 4, HB, 1024), jnp.float32),
            pltpu.SemaphoreType.DMA((2,)),
            pltpu.SemaphoreType.DMA((2,)),
        ],
        compiler_params=pltpu.CompilerParams(use_tc_tiling_on_sc=False, needs_layout_passes=False),
    )
    def body(idx_hbm, table_hbm, out_hbm, idx_v, rows_v, trans_v, gsem, osem):
        wid = lax.axis_index("s") * NC + lax.axis_index("c")
        jb0 = wid * JPW

        iota = lax.iota(jnp.int32, 16)
        i_lo = iota >> 3
        i_hi = i_lo + 2
        w_base = (iota & 7) * 128

        def wait_write(hf):
            pltpu.make_async_copy(
                trans_v.at[hf],
                out_hbm.at[pl.ds(0, 4), :, pl.ds(0, 1024)],
                osem.at[hf],
            ).wait()

        def tblock(s, carry):
            jb = jb0 + s // TR
            tr = s % TR
            t0 = tr * 8
            pltpu.sync_copy(idx_hbm.at[tr, jb], idx_v)
            copies = [
                [
                    pltpu.async_copy(
                        table_hbm.at[idx_v.at[hf * 4 + u]],
                        rows_v.at[hf, u],
                        gsem.at[hf],
                    )
                    for u in range(4)
                ]
                for hf in range(2)
            ]
            for hf in range(2):
                for c in copies[hf]:
                    c.wait()
                @pl.when(s >= 1)
                def _():
                    wait_write(hf)

                def bloop(b0, c2):
                    for u in range(4):
                        for k in range(8):
                            b = b0 * 8 + k
                            bv = w_base + b
                            x0 = rows_v[hf, u, b, pl.ds(0, 16)]
                            x1 = rows_v[hf, u, b, pl.ds(16, 16)]
                            plsc.store_scatter(trans_v.at[hf, u], [i_lo, bv], x0)
                            plsc.store_scatter(trans_v.at[hf, u], [i_hi, bv], x1)
                    return c2

                lax.fori_loop(0, 16, bloop, 0)
                pltpu.async_copy(
                    trans_v.at[hf],
                    out_hbm.at[pl.ds(t0 + hf * 4, 4), :, pl.ds(jb * 1024, 1024)],
                    osem.at[hf],
                )
            return carry

        lax.fori_loop(0, JPW * TR, tblock, 0)
        for hf in range(2):
            wait_write(hf)

    return body(idx4, table)


def kernel(input_ids, W_embed):
    idx4 = input_ids.T.reshape(TR, 8, JB, 128).transpose(0, 2, 1, 3)
    out2 = _embed_lookup(idx4, W_embed)
    out = (
        out2.reshape(T, HB, JB, 8, 128)
        .transpose(2, 4, 0, 1, 3)
        .reshape(B, T, H)
    )
    return out


# speedup vs baseline: 3.2580x; 3.2580x over previous
import functools

import jax
import jax.numpy as jnp
from jax import lax
from jax.experimental import pallas as pl
from jax.experimental.pallas import tpu as pltpu
from jax.experimental.pallas import tpu_sc as plsc

NC = 2
NS = 16
NW = NC * NS

T = 200
B = 16384
H = 32
TR = T // 8
JB = B // 128
JPW = JB // NW
HB = H // 8


@jax.jit
def _embed_lookup(idx4, table):
    mesh = plsc.VectorSubcoreMesh(core_axis_name="c", subcore_axis_name="s")

    @functools.partial(
        pl.kernel,
        out_type=jax.ShapeDtypeStruct((T, HB, JB * 1024), jnp.float32),
        mesh=mesh,
        scratch_types=[
            pltpu.VMEM((8, 128), jnp.int32),
            pltpu.VMEM((2, 4, 128, H), jnp.float32),
            pltpu.VMEM((2, 4, HB, 1024), jnp.float32),
            pltpu.SemaphoreType.DMA((2,)),
            pltpu.SemaphoreType.DMA((2,)),
        ],
        compiler_params=pltpu.CompilerParams(use_tc_tiling_on_sc=False, needs_layout_passes=False),
    )
    def body(idx_hbm, table_hbm, out_hbm, idx_v, rows_v, trans_v, gsem, osem):
        wid = lax.axis_index("s") * NC + lax.axis_index("c")
        jb0 = wid * JPW

        iota = lax.iota(jnp.int32, 16)

        def wait_write(hf):
            pltpu.make_async_copy(
                trans_v.at[hf],
                out_hbm.at[pl.ds(0, 4), :, pl.ds(0, 1024)],
                osem.at[hf],
            ).wait()

        def tblock(s, carry):
            jb = jb0 + s // TR
            tr = s % TR
            t0 = tr * 8
            pltpu.sync_copy(idx_hbm.at[tr, jb], idx_v)
            copies = [
                [
                    pltpu.async_copy(
                        table_hbm.at[idx_v.at[hf * 4 + u]],
                        rows_v.at[hf, u],
                        gsem.at[hf],
                    )
                    for u in range(4)
                ]
                for hf in range(2)
            ]
            for hf in range(2):
                for c in copies[hf]:
                    c.wait()
                @pl.when(s >= 1)
                def _():
                    wait_write(hf)

                @functools.partial(plsc.parallel_loop, 0, 8, unroll=2)
                def _(b0):
                    bvec = iota + b0 * 16
                    for u in range(4):
                        for h in range(H):
                            hv = jnp.full((16,), h, jnp.int32)
                            x = plsc.load_gather(rows_v.at[hf, u], [bvec, hv])
                            trans_v[hf, u, h >> 3,
                                    pl.ds((h & 7) * 128 + b0 * 16, 16)] = x
                pltpu.async_copy(
                    trans_v.at[hf],
                    out_hbm.at[pl.ds(t0 + hf * 4, 4), :, pl.ds(jb * 1024, 1024)],
                    osem.at[hf],
                )
            return carry

        lax.fori_loop(0, JPW * TR, tblock, 0)
        for hf in range(2):
            wait_write(hf)

    return body(idx4, table)


def kernel(input_ids, W_embed):
    idx4 = input_ids.T.reshape(TR, 8, JB, 128).transpose(0, 2, 1, 3)
    out2 = _embed_lookup(idx4, W_embed)
    out = (
        out2.reshape(T, HB, JB, 8, 128)
        .transpose(2, 4, 0, 1, 3)
        .reshape(B, T, H)
    )
    return out
